# baseline (speedup 1.0000x reference)
import functools

import jax
import jax.numpy as jnp
from jax import lax
from jax.experimental import pallas as pl
from jax.experimental.pallas import tpu as pltpu
from jax.experimental.pallas import tpu_sc as plsc

_HUGE = 1e31
_EPS = 2.220446049250313e-16

_B, _A, _D = 1024, 200, 128
_AD = 3 * _D
_L = 16
_NC, _NS = 2, 16
_NW = _NC * _NS
_RPW = _B // _NW
_CH = 50
_NCH = _A // _CH
_AP = 208
_PF = 256


def _mlp_body(obs_ref, w1_ref, b1_ref, w2_ref, b2_ref, x2_ref):
    x = lax.dot_general(obs_ref[...], w1_ref[...], (((1,), (1,)), ((), ())),
                        preferred_element_type=jnp.float32,
                        precision=lax.Precision.HIGHEST)
    x = jnp.maximum(x + b1_ref[...], 0.0)
    x2 = lax.dot_general(x, w2_ref[...], (((1,), (1,)), ((), ())),
                         preferred_element_type=jnp.float32,
                         precision=lax.Precision.HIGHEST)
    x2_ref[...] = x2 + b2_ref[...]


_mlp_call = pl.pallas_call(
    _mlp_body,
    out_shape=jax.ShapeDtypeStruct((_B, _AD), jnp.float32),
)


def _score_body(x2_hbm, idx_hbm, rel_hbm, ent_hbm, tri_hbm, out_hbm,
                idx0, idx1, x2_0, x2_1, rows_v, sc0, sc1,
                sem_g, sem_i0, sem_i1, sem_w0, sem_w1):
    wid = lax.axis_index("s") * _NC + lax.axis_index("c")
    iota = lax.iota(jnp.int32, _L)
    base = wid * _RPW
    tables = (rel_hbm, ent_hbm, tri_hbm)
    idx_slots = (idx0, idx1)
    x2_slots = (x2_0, x2_1)
    sc_slots = (sc0, sc1)
    sem_i = (sem_i0, sem_i1)
    sem_w = (sem_w0, sem_w1)

    def idx_cps(r, s):
        return (pltpu.make_async_copy(idx_hbm.at[r], idx_slots[s], sem_i[s]),
                pltpu.make_async_copy(x2_hbm.at[r], x2_slots[s], sem_i[s]))

    def g_cps(s):
        return tuple(
            pltpu.make_async_copy(tables[t].at[idx_slots[s].at[t, j]],
                                  rows_v.at[pl.ds((t * _NCH + j) * _CH, _CH)],
                                  sem_g)
            for t in range(3) for j in range(_NCH))

    def wb_cp(r, s):
        return pltpu.make_async_copy(sc_slots[s], out_hbm.at[r], sem_w[s])

    for cp in idx_cps(base, 0):
        cp.start()
    for cp in idx_cps(base, 0):
        cp.wait()
    for cp in idx_cps(base + 1, 1):
        cp.start()

    def pair_body(i, carry):
        not_first = i > 0
        not_last = i < _RPW // 2 - 1
        for s in range(2):
            r = base + 2 * i + s
            if s == 0:
                @pl.when(not_first)
                def _():
                    for cp in idx_cps(r, 0):
                        cp.wait()
            else:
                for cp in idx_cps(r, 1):
                    cp.wait()
            for cp in g_cps(s):
                cp.start()
            for cp in g_cps(s):
                cp.wait()
            x2c = [x2_slots[s][pl.ds(k * _L, _L)] for k in range(_AD // _L)]

            @pl.when(not_last)
            def _():
                for cp in idx_cps(r + 2, s):
                    cp.start()

            @pl.when(not_first)
            def _():
                wb_cp(r, s).wait()

            scr = sc_slots[s]
            lane0 = iota == 0
            zvec = jnp.zeros((_L,), jnp.int32)
            perms = [jnp.bitwise_and(iota + (1 << p), _L - 1)
                     for p in (3, 2, 1, 0)]

            @plsc.parallel_loop(0, _AP, unroll=2)
            def action_body(a):
                accs = [None, None, None, None]
                for t in range(3):
                    for k in range(_D // _L):
                        v = rows_v[(t * _NCH) * _CH + a, pl.ds(k * _L, _L)]
                        i2 = t * (_D // _L) + k
                        p = v * x2c[i2]
                        accs[i2 % 4] = (p if accs[i2 % 4] is None
                                        else accs[i2 % 4] + p)
                acc = (accs[0] + accs[1]) + (accs[2] + accs[3])
                for pv in perms:
                    acc = acc + lax.gather(
                        acc, pv[:, None],
                        lax.GatherDimensionNumbers(
                            offset_dims=(), collapsed_slice_dims=(0,),
                            start_index_map=(0,)),
                        slice_sizes=(1,),
                        mode=lax.GatherScatterMode.PROMISE_IN_BOUNDS)
                plsc.store_scatter(scr, [a + zvec], acc, mask=lane0)

            wb_cp(r, s).start()
        return carry

    lax.fori_loop(0, _RPW // 2, pair_body, 0)
    wb_cp(base + _RPW - 2, 0).wait()
    wb_cp(base + _RPW - 1, 1).wait()


_score_call = functools.partial(
    pl.kernel,
    out_type=jax.ShapeDtypeStruct((_B, _PF), jnp.float32),
    mesh=plsc.VectorSubcoreMesh(core_axis_name="c", subcore_axis_name="s"),
    compiler_params=pltpu.CompilerParams(needs_layout_passes=False),
    scratch_types=[
        pltpu.VMEM((3, _NCH, _CH), jnp.int32),
        pltpu.VMEM((3, _NCH, _CH), jnp.int32),
        pltpu.VMEM((_AD,), jnp.float32),
        pltpu.VMEM((_AD,), jnp.float32),
        pltpu.VMEM((3 * _AP, _D), jnp.float32),
        pltpu.VMEM((_PF,), jnp.float32),
        pltpu.VMEM((_PF,), jnp.float32),
        pltpu.SemaphoreType.DMA,
        pltpu.SemaphoreType.DMA,
        pltpu.SemaphoreType.DMA,
        pltpu.SemaphoreType.DMA,
        pltpu.SemaphoreType.DMA,
    ],
)(_score_body)


def _smx_body(part_ref, mask_ref, p_ref, ent_ref):
    s = part_ref[:, :_A]
    s = s - (1.0 - mask_ref[...]) * _HUGE
    m = jnp.max(s, axis=1, keepdims=True)
    e = jnp.exp(s - m)
    z = jnp.sum(e, axis=1, keepdims=True)
    p = e / z
    p_ref[...] = p
    ent_ref[...] = jnp.sum(-p * jnp.log(p + _EPS), axis=1, keepdims=True)


_smx_call = pl.pallas_call(
    _smx_body,
    out_shape=(jax.ShapeDtypeStruct((_B, _A), jnp.float32),
               jax.ShapeDtypeStruct((_B, 1), jnp.float32)),
)


def kernel(obs, r_space, e_space, triple_id, action_mask,
           W1_w, W1_b, W2_w, W2_b, rel_table, ent_table, triple_table):
    x2 = _mlp_call(obs, W1_w, W1_b.reshape(1, _AD), W2_w, W2_b.reshape(1, _AD))
    idx = jnp.stack(
        [r_space.astype(jnp.int32), e_space.astype(jnp.int32),
         triple_id.astype(jnp.int32)], axis=1).reshape(_B, 3, _NCH, _CH)
    part = _score_call(x2, idx, rel_table, ent_table, triple_table)
    p, ent_col = _smx_call(part, action_mask)
    return (p, ent_col.reshape(_B))


# speedup vs baseline: 1.0465x; 1.0465x over previous
import functools

import jax
import jax.numpy as jnp
from jax import lax
from jax.experimental import pallas as pl
from jax.experimental.pallas import tpu as pltpu
from jax.experimental.pallas import tpu_sc as plsc

_HUGE = 1e31
_EPS = 2.220446049250313e-16

_B, _A, _D = 1024, 200, 128
_AD = 3 * _D
_L = 16
_NC, _NS = 2, 16
_NW = _NC * _NS
_RPW = _B // _NW
_CH = 100
_NCH = _A // _CH
_AP = 208
_PF = 256


def _mlp_body(obs_ref, w1_ref, b1_ref, w2_ref, b2_ref, x2_ref):
    x = lax.dot_general(obs_ref[...], w1_ref[...], (((1,), (1,)), ((), ())),
                        preferred_element_type=jnp.float32)
    x = jnp.maximum(x + b1_ref[...], 0.0)
    x2 = lax.dot_general(x, w2_ref[...], (((1,), (1,)), ((), ())),
                         preferred_element_type=jnp.float32)
    x2_ref[...] = x2 + b2_ref[...]


_mlp_call = pl.pallas_call(
    _mlp_body,
    out_shape=jax.ShapeDtypeStruct((_B, _AD), jnp.float32),
)


def _score_body(x2_hbm, idx_hbm, rel_hbm, ent_hbm, tri_hbm, out_hbm,
                idx0, idx1, x2_0, x2_1, rows_v, sc0, sc1,
                sem_g, sem_i0, sem_i1, sem_w0, sem_w1):
    wid = lax.axis_index("s") * _NC + lax.axis_index("c")
    iota = lax.iota(jnp.int32, _L)
    base = wid * _RPW
    tables = (rel_hbm, ent_hbm, tri_hbm)
    idx_slots = (idx0, idx1)
    x2_slots = (x2_0, x2_1)
    sc_slots = (sc0, sc1)
    sem_i = (sem_i0, sem_i1)
    sem_w = (sem_w0, sem_w1)

    def idx_cps(r, s):
        return (pltpu.make_async_copy(idx_hbm.at[r], idx_slots[s], sem_i[s]),
                pltpu.make_async_copy(x2_hbm.at[r], x2_slots[s], sem_i[s]))

    def g_cps(s):
        return tuple(
            pltpu.make_async_copy(tables[t].at[idx_slots[s].at[t, j]],
                                  rows_v.at[pl.ds((t * _NCH + j) * _CH, _CH)],
                                  sem_g)
            for t in range(3) for j in range(_NCH))

    def wb_cp(r, s):
        return pltpu.make_async_copy(sc_slots[s], out_hbm.at[r], sem_w[s])

    for cp in idx_cps(base, 0):
        cp.start()
    for cp in idx_cps(base, 0):
        cp.wait()
    for cp in idx_cps(base + 1, 1):
        cp.start()

    def pair_body(i, carry):
        not_first = i > 0
        not_last = i < _RPW // 2 - 1
        for s in range(2):
            r = base + 2 * i + s
            if s == 0:
                @pl.when(not_first)
                def _():
                    for cp in idx_cps(r, 0):
                        cp.wait()
            else:
                for cp in idx_cps(r, 1):
                    cp.wait()
            for cp in g_cps(s):
                cp.start()
            for cp in g_cps(s):
                cp.wait()
            x2c = [x2_slots[s][pl.ds(k * _L, _L)] for k in range(_AD // _L)]

            @pl.when(not_last)
            def _():
                for cp in idx_cps(r + 2, s):
                    cp.start()

            @pl.when(not_first)
            def _():
                wb_cp(r, s).wait()

            scr = sc_slots[s]
            lane0 = iota == 0
            zvec = jnp.zeros((_L,), jnp.int32)
            perms = [jnp.bitwise_and(iota + (1 << p), _L - 1)
                     for p in (3, 2, 1, 0)]

            @plsc.parallel_loop(0, _AP, unroll=2)
            def action_body(a):
                accs = [None, None, None, None]
                for t in range(3):
                    for k in range(_D // _L):
                        v = rows_v[(t * _NCH) * _CH + a, pl.ds(k * _L, _L)]
                        i2 = t * (_D // _L) + k
                        p = v * x2c[i2]
                        accs[i2 % 4] = (p if accs[i2 % 4] is None
                                        else accs[i2 % 4] + p)
                acc = (accs[0] + accs[1]) + (accs[2] + accs[3])
                for pv in perms:
                    acc = acc + lax.gather(
                        acc, pv[:, None],
                        lax.GatherDimensionNumbers(
                            offset_dims=(), collapsed_slice_dims=(0,),
                            start_index_map=(0,)),
                        slice_sizes=(1,),
                        mode=lax.GatherScatterMode.PROMISE_IN_BOUNDS)
                plsc.store_scatter(scr, [a + zvec], acc, mask=lane0)

            wb_cp(r, s).start()
        return carry

    lax.fori_loop(0, _RPW // 2, pair_body, 0)
    wb_cp(base + _RPW - 2, 0).wait()
    wb_cp(base + _RPW - 1, 1).wait()


_score_call = functools.partial(
    pl.kernel,
    out_type=jax.ShapeDtypeStruct((_B, _PF), jnp.float32),
    mesh=plsc.VectorSubcoreMesh(core_axis_name="c", subcore_axis_name="s"),
    compiler_params=pltpu.CompilerParams(needs_layout_passes=False),
    scratch_types=[
        pltpu.VMEM((3, _NCH, _CH), jnp.int32),
        pltpu.VMEM((3, _NCH, _CH), jnp.int32),
        pltpu.VMEM((_AD,), jnp.float32),
        pltpu.VMEM((_AD,), jnp.float32),
        pltpu.VMEM((3 * _AP, _D), jnp.float32),
        pltpu.VMEM((_PF,), jnp.float32),
        pltpu.VMEM((_PF,), jnp.float32),
        pltpu.SemaphoreType.DMA,
        pltpu.SemaphoreType.DMA,
        pltpu.SemaphoreType.DMA,
        pltpu.SemaphoreType.DMA,
        pltpu.SemaphoreType.DMA,
    ],
)(_score_body)


def _smx_body(part_ref, mask_ref, p_ref, ent_ref):
    s = part_ref[:, :_A]
    s = s - (1.0 - mask_ref[...]) * _HUGE
    m = jnp.max(s, axis=1, keepdims=True)
    e = jnp.exp(s - m)
    z = jnp.sum(e, axis=1, keepdims=True)
    p = e / z
    p_ref[...] = p
    ent_ref[...] = jnp.sum(-p * jnp.log(p + _EPS), axis=1, keepdims=True)


_smx_call = pl.pallas_call(
    _smx_body,
    out_shape=(jax.ShapeDtypeStruct((_B, _A), jnp.float32),
               jax.ShapeDtypeStruct((_B, 1), jnp.float32)),
)


def kernel(obs, r_space, e_space, triple_id, action_mask,
           W1_w, W1_b, W2_w, W2_b, rel_table, ent_table, triple_table):
    x2 = _mlp_call(obs, W1_w, W1_b.reshape(1, _AD), W2_w, W2_b.reshape(1, _AD))
    idx = jnp.stack(
        [r_space.astype(jnp.int32), e_space.astype(jnp.int32),
         triple_id.astype(jnp.int32)], axis=1).reshape(_B, 3, _NCH, _CH)
    part = _score_call(x2, idx, rel_table, ent_table, triple_table)
    p, ent_col = _smx_call(part, action_mask)
    return (p, ent_col.reshape(_B))


# speedup vs baseline: 1.0635x; 1.0163x over previous
import functools

import jax
import jax.numpy as jnp
from jax import lax
from jax.experimental import pallas as pl
from jax.experimental.pallas import tpu as pltpu
from jax.experimental.pallas import tpu_sc as plsc

_HUGE = 1e31
_EPS = 2.220446049250313e-16

_B, _A, _D = 1024, 200, 128
_AD = 3 * _D
_L = 16
_NC, _NS = 2, 16
_NW = _NC * _NS
_RPW = _B // _NW
_CH = 100
_NCH = _A // _CH
_AP = 208
_PF = 256


def _mlp_body(obs_ref, w1_ref, b1_ref, w2_ref, b2_ref, x2_ref):
    x = lax.dot_general(obs_ref[...], w1_ref[...], (((1,), (1,)), ((), ())),
                        preferred_element_type=jnp.float32)
    x = jnp.maximum(x + b1_ref[...], 0.0)
    x2 = lax.dot_general(x, w2_ref[...], (((1,), (1,)), ((), ())),
                         preferred_element_type=jnp.float32)
    x2_ref[...] = x2 + b2_ref[...]


_mlp_call = pl.pallas_call(
    _mlp_body,
    out_shape=jax.ShapeDtypeStruct((_B, _AD), jnp.float32),
)


def _score_body(x2_hbm, idx_hbm, rel_hbm, ent_hbm, tri_hbm, out_hbm,
                idx0, idx1, x2_0, x2_1, rows_v, sc0, sc1,
                sem_g, sem_i0, sem_i1, sem_w0, sem_w1):
    wid = lax.axis_index("s") * _NC + lax.axis_index("c")
    iota = lax.iota(jnp.int32, _L)
    base = wid * _RPW
    tables = (rel_hbm, ent_hbm, tri_hbm)
    idx_slots = (idx0, idx1)
    x2_slots = (x2_0, x2_1)
    sc_slots = (sc0, sc1)
    sem_i = (sem_i0, sem_i1)
    sem_w = (sem_w0, sem_w1)

    def idx_cps(r, s):
        return (pltpu.make_async_copy(idx_hbm.at[r], idx_slots[s], sem_i[s]),
                pltpu.make_async_copy(x2_hbm.at[r], x2_slots[s], sem_i[s]))

    def g_cps(s):
        return tuple(
            pltpu.make_async_copy(tables[t].at[idx_slots[s].at[t, j]],
                                  rows_v.at[pl.ds((t * _NCH + j) * _CH, _CH)],
                                  sem_g)
            for t in range(3) for j in range(_NCH))

    def wb_cp(r, s):
        return pltpu.make_async_copy(sc_slots[s], out_hbm.at[r], sem_w[s])

    for cp in idx_cps(base, 0):
        cp.start()
    for cp in idx_cps(base, 0):
        cp.wait()
    for cp in idx_cps(base + 1, 1):
        cp.start()

    def pair_body(i, carry):
        not_first = i > 0
        not_last = i < _RPW // 2 - 1
        for s in range(2):
            r = base + 2 * i + s
            if s == 0:
                @pl.when(not_first)
                def _():
                    for cp in idx_cps(r, 0):
                        cp.wait()
            else:
                for cp in idx_cps(r, 1):
                    cp.wait()
            for cp in g_cps(s):
                cp.start()
            for cp in g_cps(s):
                cp.wait()
            x2c = [x2_slots[s][pl.ds(k * _L, _L)] for k in range(_AD // _L)]

            @pl.when(not_last)
            def _():
                for cp in idx_cps(r + 2, s):
                    cp.start()

            @pl.when(not_first)
            def _():
                wb_cp(r, s).wait()

            scr = sc_slots[s]
            lane0 = iota == 0
            zvec = jnp.zeros((_L,), jnp.int32)
            perms = [jnp.bitwise_and(iota + (1 << p), _L - 1)
                     for p in (3, 2, 1, 0)]

            @plsc.parallel_loop(0, _A, unroll=2)
            def action_body(a):
                accs = [None, None, None, None]
                for t in range(3):
                    for k in range(_D // _L):
                        v = rows_v[(t * _NCH) * _CH + a, pl.ds(k * _L, _L)]
                        i2 = t * (_D // _L) + k
                        p = v * x2c[i2]
                        accs[i2 % 4] = (p if accs[i2 % 4] is None
                                        else accs[i2 % 4] + p)
                acc = (accs[0] + accs[1]) + (accs[2] + accs[3])
                for pv in perms:
                    acc = acc + lax.gather(
                        acc, pv[:, None],
                        lax.GatherDimensionNumbers(
                            offset_dims=(), collapsed_slice_dims=(0,),
                            start_index_map=(0,)),
                        slice_sizes=(1,),
                        mode=lax.GatherScatterMode.PROMISE_IN_BOUNDS)
                plsc.store_scatter(scr, [a + zvec], acc, mask=lane0)

            wb_cp(r, s).start()
        return carry

    lax.fori_loop(0, _RPW // 2, pair_body, 0)
    wb_cp(base + _RPW - 2, 0).wait()
    wb_cp(base + _RPW - 1, 1).wait()


_score_call = functools.partial(
    pl.kernel,
    out_type=jax.ShapeDtypeStruct((_B, _PF), jnp.float32),
    mesh=plsc.VectorSubcoreMesh(core_axis_name="c", subcore_axis_name="s"),
    compiler_params=pltpu.CompilerParams(needs_layout_passes=False),
    scratch_types=[
        pltpu.VMEM((3, _NCH, _CH), jnp.int32),
        pltpu.VMEM((3, _NCH, _CH), jnp.int32),
        pltpu.VMEM((_AD,), jnp.float32),
        pltpu.VMEM((_AD,), jnp.float32),
        pltpu.VMEM((3 * _A, _D), jnp.float32),
        pltpu.VMEM((_PF,), jnp.float32),
        pltpu.VMEM((_PF,), jnp.float32),
        pltpu.SemaphoreType.DMA,
        pltpu.SemaphoreType.DMA,
        pltpu.SemaphoreType.DMA,
        pltpu.SemaphoreType.DMA,
        pltpu.SemaphoreType.DMA,
    ],
)(_score_body)


def _smx_body(part_ref, mask_ref, p_ref, ent_ref):
    s = part_ref[:, :_A]
    s = s - (1.0 - mask_ref[...]) * _HUGE
    m = jnp.max(s, axis=1, keepdims=True)
    e = jnp.exp(s - m)
    z = jnp.sum(e, axis=1, keepdims=True)
    p = e / z
    p_ref[...] = p
    ent_ref[...] = jnp.sum(-p * jnp.log(p + _EPS), axis=1, keepdims=True)


_smx_call = pl.pallas_call(
    _smx_body,
    out_shape=(jax.ShapeDtypeStruct((_B, _A), jnp.float32),
               jax.ShapeDtypeStruct((_B, 1), jnp.float32)),
)


def kernel(obs, r_space, e_space, triple_id, action_mask,
           W1_w, W1_b, W2_w, W2_b, rel_table, ent_table, triple_table):
    x2 = _mlp_call(obs, W1_w, W1_b.reshape(1, _AD), W2_w, W2_b.reshape(1, _AD))
    idx = jnp.stack(
        [r_space.astype(jnp.int32), e_space.astype(jnp.int32),
         triple_id.astype(jnp.int32)], axis=1).reshape(_B, 3, _NCH, _CH)
    part = _score_call(x2, idx, rel_table, ent_table, triple_table)
    p, ent_col = _smx_call(part, action_mask)
    return (p, ent_col.reshape(_B))
